# baseline (speedup 1.0000x reference)
import functools

import jax
import jax.numpy as jnp
from jax import lax
from jax.experimental import pallas as pl
from jax.experimental.pallas import tpu as pltpu
from jax.experimental.pallas import tpu_sc as plsc

N_NODES = 10000
N_EDGES = 320000
H = 128
DE = 16

NC = 2
NS = 16
NW = NC * NS
LANE = 16

NPAD = 10240
EPAD = 327680
EPW = EPAD // NW
CHA = 64
NCHA = EPW // CHA
CHB = 128
NCHB = EPW // CHB
NEG = -3.0e38

_sc_mesh = plsc.VectorSubcoreMesh(
    core_axis_name="c", subcore_axis_name="s", num_cores=NC, num_subcores=NS)

_sc_params = pltpu.CompilerParams(needs_layout_passes=False)


def _node_mm_body(x_ref, wq_ref, bq_ref, wn_ref, bn_ref, q_ref, kn_ref):
    x = x_ref[...]
    q_ref[...] = jnp.dot(x, wq_ref[...], preferred_element_type=jnp.float32) + bq_ref[...]
    kn_ref[...] = jnp.dot(x, wn_ref[...], preferred_element_type=jnp.float32) + bn_ref[...]


def _node_mm(xp, Wq, bq, Wn, bn):
    blk = 1024
    grid = NPAD // blk
    return pl.pallas_call(
        _node_mm_body,
        grid=(grid,),
        in_specs=[
            pl.BlockSpec((blk, H), lambda i: (i, 0)),
            pl.BlockSpec((H, H), lambda i: (0, 0)),
            pl.BlockSpec((1, H), lambda i: (0, 0)),
            pl.BlockSpec((H, H), lambda i: (0, 0)),
            pl.BlockSpec((1, H), lambda i: (0, 0)),
        ],
        out_specs=[
            pl.BlockSpec((blk, H), lambda i: (i, 0)),
            pl.BlockSpec((blk, H), lambda i: (i, 0)),
        ],
        out_shape=[
            jax.ShapeDtypeStruct((NPAD, H), jnp.float32),
            jax.ShapeDtypeStruct((NPAD, H), jnp.float32),
        ],
    )(xp, Wq, bq.reshape(1, H), Wn, bn.reshape(1, H))


def _edge_mm_body(ea_ref, we_ref, be_ref, ke_ref):
    ke_ref[...] = (jnp.dot(ea_ref[...], we_ref[...],
                           preferred_element_type=jnp.float32) + be_ref[...])


def _edge_mm(eap, We, be):
    blk = 4096
    grid = EPAD // blk
    return pl.pallas_call(
        _edge_mm_body,
        grid=(grid,),
        in_specs=[
            pl.BlockSpec((blk, DE), lambda i: (i, 0)),
            pl.BlockSpec((DE, H), lambda i: (0, 0)),
            pl.BlockSpec((1, H), lambda i: (0, 0)),
        ],
        out_specs=pl.BlockSpec((blk, H), lambda i: (i, 0)),
        out_shape=jax.ShapeDtypeStruct((EPAD, H), jnp.float32),
    )(eap, We, be.reshape(1, H))


def _combine_max_body(mall_ref, m_ref):
    m = jnp.max(mall_ref[...], axis=0, keepdims=True)
    m_ref[...] = jnp.where(m > NEG * 0.5, m, 0.0)


def _combine_max(m_all):
    return pl.pallas_call(
        _combine_max_body,
        out_shape=jax.ShapeDtypeStruct((1, NPAD), jnp.float32),
    )(m_all)


def _combine_recip_body(dall_ref, r_ref):
    d = jnp.sum(dall_ref[...], axis=0, keepdims=True)
    r_ref[...] = 1.0 / (d + 1e-9)


def _combine_recip(d_all):
    return pl.pallas_call(
        _combine_recip_body,
        out_shape=jax.ShapeDtypeStruct((1, NPAD), jnp.float32),
    )(d_all)


def _norm_relu_body(acc_ref, r_ref, out_ref):
    out_ref[...] = jnp.maximum((acc_ref[0] + acc_ref[1]) * r_ref[...], 0.0)


def _norm_relu(acc, recip_col):
    blk = 1024
    grid = NPAD // blk
    return pl.pallas_call(
        _norm_relu_body,
        grid=(grid,),
        in_specs=[
            pl.BlockSpec((NC, blk, H), lambda i: (0, i, 0)),
            pl.BlockSpec((blk, 1), lambda i: (i, 0)),
        ],
        out_specs=pl.BlockSpec((blk, H), lambda i: (i, 0)),
        out_shape=jax.ShapeDtypeStruct((NPAD, H), jnp.float32),
    )(acc, recip_col)


def _wid():
    return lax.axis_index("s") * NC + lax.axis_index("c")


def _scatter_max(tbl, dst16, val16):
    def _pending(_):
        cur = plsc.load_gather(tbl, [dst16])
        return plsc.all_reduce_population_count(val16 > cur)[0] > 0

    def _body(_):
        cur = plsc.load_gather(tbl, [dst16])
        plsc.store_scatter(tbl, [dst16], jnp.maximum(cur, val16),
                           mask=val16 > cur)
        return 0

    lax.while_loop(_pending, _body, 0)


def _sc_pass_a(ei_h, kn_h, q_h, ke_h, a_h,
               kv_h, lg_h, mall_h,
               a_v, ei_v, kn_v, q_v, ke_v, kv_v, lg_v, m_tbl,
               sem_i, sem_g0, sem_g1, sem_w0, sem_w1):
    w = _wid()
    base0 = w * EPW
    sem_g = (sem_g0, sem_g1)
    sem_w = (sem_w0, sem_w1)
    pltpu.sync_copy(a_h, a_v)

    @pl.loop(0, NPAD // LANE)
    def _init(i):
        m_tbl[pl.ds(i * LANE, LANE)] = jnp.full((LANE,), NEG, jnp.float32)

    def _issue_gathers(ci, b):
        base = base0 + ci * CHA
        pltpu.async_copy(kn_h.at[ei_v.at[b, 0]], kn_v.at[b], sem_g[b])
        pltpu.async_copy(q_h.at[ei_v.at[b, 1]], q_v.at[b], sem_g[b])
        pltpu.async_copy(ke_h.at[pl.ds(base, CHA)], ke_v.at[b], sem_g[b])

    def _wait_gathers(b):
        pltpu.make_async_copy(kn_h.at[pl.ds(0, CHA)], kn_v.at[b], sem_g[b]).wait()
        pltpu.make_async_copy(q_h.at[pl.ds(0, CHA)], q_v.at[b], sem_g[b]).wait()
        pltpu.make_async_copy(ke_h.at[pl.ds(0, CHA)], ke_v.at[b], sem_g[b]).wait()

    def _wait_wb(b):
        pltpu.make_async_copy(kv_v.at[b], kv_h.at[pl.ds(0, CHA)], sem_w[b]).wait()
        pltpu.make_async_copy(lg_v.at[b], lg_h.at[pl.ds(0, CHA)], sem_w[b]).wait()

    def _compute(ci, b):
        base = base0 + ci * CHA

        @pl.loop(0, CHA // LANE)
        def _grp(g):
            lanes = lax.iota(jnp.int32, LANE)
            lg16 = jnp.zeros((LANE,), jnp.float32)
            for k in range(LANE):
                e = g * LANE + k
                acc = jnp.zeros((LANE,), jnp.float32)
                for gg in range(H // LANE):
                    sl = pl.ds(gg * LANE, LANE)
                    kv = kn_v[b, e, sl] + ke_v[b, e, sl]
                    kv_v[b, e, sl] = kv
                    u = q_v[b, e, sl] + kv
                    f = jnp.where(u > 0.0, u, 0.2 * u)
                    acc = acc + f * a_v[sl]
                lg16 = jnp.where(lanes == k, jnp.sum(acc), lg16)
            lg_v[b, pl.ds(g * LANE, LANE)] = lg16
            dst16 = ei_v[b, 1, pl.ds(g * LANE, LANE)]
            _scatter_max(m_tbl, dst16, lg16)

        pltpu.async_copy(kv_v.at[b], kv_h.at[pl.ds(base, CHA)], sem_w[b])
        pltpu.async_copy(lg_v.at[b], lg_h.at[pl.ds(base, CHA)], sem_w[b])

    pltpu.sync_copy(ei_h.at[w * NCHA], ei_v.at[0])
    _issue_gathers(0, 0)

    @pl.loop(0, NCHA // 2)
    def _outer(g2):
        for b in (0, 1):
            ci = g2 * 2 + b

            @pl.when(ci + 1 < NCHA)
            def _pf():
                nb = 1 - b
                pltpu.sync_copy(ei_h.at[w * NCHA + ci + 1], ei_v.at[nb])

            @pl.when(ci >= 2)
            def _ww():
                _wait_wb(b)

            _wait_gathers(b)

            @pl.when(ci + 1 < NCHA)
            def _ig():
                _issue_gathers(ci + 1, 1 - b)

            _compute(ci, b)

    _wait_wb(0)
    _wait_wb(1)
    pltpu.sync_copy(m_tbl, mall_h.at[w])


def _sc_pass_b(dst_h, ev_h, kv_h,
               acc_h,
               dst_v, ev_v, kv_v, acc_sh,
               sem_s0, sem_s1, sem_a0, sem_a1):
    c = lax.axis_index("c")
    s = lax.axis_index("s")
    w = _wid()
    base0 = w * EPW
    rows_per_tile = NPAD // NS
    sem_s = (sem_s0, sem_s1)
    sem_a = (sem_a0, sem_a1)

    @pl.loop(0, CHB)
    def _z(e):
        for g in range(H // LANE):
            kv_v[0, e, pl.ds(g * LANE, LANE)] = jnp.zeros((LANE,), jnp.float32)

    for j in range(rows_per_tile // CHB):
        pltpu.sync_copy(kv_v.at[0],
                        acc_sh.at[pl.ds(s * rows_per_tile + j * CHB, CHB)])

    plsc.subcore_barrier()

    def _issue_streams(ci, b):
        base = base0 + ci * CHB
        pltpu.async_copy(dst_h.at[pl.ds(base, CHB)], dst_v.at[b], sem_s[b])
        pltpu.async_copy(ev_h.at[pl.ds(base, CHB)], ev_v.at[b], sem_s[b])
        pltpu.async_copy(kv_h.at[pl.ds(base, CHB)], kv_v.at[b], sem_s[b])

    def _wait_streams(b):
        pltpu.make_async_copy(dst_h.at[pl.ds(0, CHB)], dst_v.at[b], sem_s[b]).wait()
        pltpu.make_async_copy(ev_h.at[pl.ds(0, CHB)], ev_v.at[b], sem_s[b]).wait()
        pltpu.make_async_copy(kv_h.at[pl.ds(0, CHB)], kv_v.at[b], sem_s[b]).wait()

    def _wait_scatter(b):
        pltpu.make_async_copy(kv_v.at[b], acc_sh.at[pl.ds(0, CHB)],
                              sem_a[b]).wait()

    _issue_streams(0, 0)

    @pl.loop(0, NCHB // 2)
    def _outer(g2):
        for b in (0, 1):
            ci = g2 * 2 + b

            @pl.when(ci >= 1)
            def _wa():
                _wait_scatter(1 - b)

            @pl.when(ci + 1 < NCHB)
            def _pf():
                _issue_streams(ci + 1, 1 - b)

            _wait_streams(b)

            @pl.loop(0, CHB // LANE)
            def _scale(g):
                ev16 = ev_v[b, pl.ds(g * LANE, LANE)]
                for k in range(LANE):
                    e = g * LANE + k
                    a_s = ev16[k]
                    for gg in range(H // LANE):
                        ssl = pl.ds(gg * LANE, LANE)
                        kv_v[b, e, ssl] = kv_v[b, e, ssl] * a_s

            pltpu.async_copy(kv_v.at[b], acc_sh.at[dst_v.at[b]], sem_a[b],
                             add=True)

    _wait_scatter((NCHB - 1) % 2)
    plsc.subcore_barrier()

    for j in range(rows_per_tile // CHB):
        r0 = s * rows_per_tile + j * CHB
        pltpu.sync_copy(acc_sh.at[pl.ds(r0, CHB)], kv_v.at[0])
        pltpu.sync_copy(kv_v.at[0], acc_h.at[c].at[pl.ds(r0, CHB)])


def _sc_pass_a2(dst_h, lg_h, m_h,
                ev_h, dall_h,
                m_tbl, den_tbl, dst_v, lg_v, ev_v):
    w = _wid()
    base0 = w * EPW
    pltpu.sync_copy(m_h, m_tbl)

    @pl.loop(0, NPAD // LANE)
    def _init(i):
        den_tbl[pl.ds(i * LANE, LANE)] = jnp.zeros((LANE,), jnp.float32)

    @pl.loop(0, NCHB)
    def _chunk(ci):
        base = base0 + ci * CHB
        pltpu.sync_copy(dst_h.at[pl.ds(base, CHB)], dst_v)
        pltpu.sync_copy(lg_h.at[pl.ds(base, CHB)], lg_v)

        @pl.loop(0, CHB // LANE)
        def _vec(g):
            sl = pl.ds(g * LANE, LANE)
            dv = dst_v[sl]
            mv = plsc.load_gather(m_tbl, [dv])
            ev = jnp.exp(lg_v[sl] - mv)
            ev_v[sl] = ev
            plsc.addupdate_scatter(den_tbl, [dv], ev)

        pltpu.sync_copy(ev_v, ev_h.at[pl.ds(base, CHB)])

    pltpu.sync_copy(den_tbl, dall_h.at[w])


_pass_a = pl.kernel(
    _sc_pass_a,
    out_type=[
        jax.ShapeDtypeStruct((EPAD, H), jnp.float32),
        jax.ShapeDtypeStruct((EPAD,), jnp.float32),
        jax.ShapeDtypeStruct((NW, NPAD), jnp.float32),
    ],
    mesh=_sc_mesh,
    compiler_params=_sc_params,
    scratch_types=[
        pltpu.VMEM((H,), jnp.float32),
        pltpu.VMEM((2, 2, CHA), jnp.int32),
        pltpu.VMEM((2, CHA, H), jnp.float32),
        pltpu.VMEM((2, CHA, H), jnp.float32),
        pltpu.VMEM((2, CHA, H), jnp.float32),
        pltpu.VMEM((2, CHA, H), jnp.float32),
        pltpu.VMEM((2, CHA), jnp.float32),
        pltpu.VMEM((NPAD,), jnp.float32),
        pltpu.SemaphoreType.DMA,
        pltpu.SemaphoreType.DMA,
        pltpu.SemaphoreType.DMA,
        pltpu.SemaphoreType.DMA,
        pltpu.SemaphoreType.DMA,
    ],
)

_pass_b = pl.kernel(
    _sc_pass_b,
    out_type=[
        jax.ShapeDtypeStruct((NC, NPAD, H), jnp.float32),
    ],
    mesh=_sc_mesh,
    compiler_params=_sc_params,
    scratch_types=[
        pltpu.VMEM((2, CHB), jnp.int32),
        pltpu.VMEM((2, CHB), jnp.float32),
        pltpu.VMEM((2, CHB, H), jnp.float32),
        pltpu.VMEM_SHARED((NPAD, H), jnp.float32),
        pltpu.SemaphoreType.DMA,
        pltpu.SemaphoreType.DMA,
        pltpu.SemaphoreType.DMA,
        pltpu.SemaphoreType.DMA,
    ],
)

_pass_a2 = pl.kernel(
    _sc_pass_a2,
    out_type=[
        jax.ShapeDtypeStruct((EPAD,), jnp.float32),
        jax.ShapeDtypeStruct((NW, NPAD), jnp.float32),
    ],
    mesh=_sc_mesh,
    compiler_params=_sc_params,
    scratch_types=[
        pltpu.VMEM((NPAD,), jnp.float32),
        pltpu.VMEM((NPAD,), jnp.float32),
        pltpu.VMEM((CHB,), jnp.int32),
        pltpu.VMEM((CHB,), jnp.float32),
        pltpu.VMEM((CHB,), jnp.float32),
    ],
)


def _hop(xp, eap, eib, dstp, Wq, bq, Wn, bn, We, be, a):
    qp, knp = _node_mm(xp, Wq, bq, Wn, bn)
    ke = _edge_mm(eap, We, be)
    kv, lg, m_all = _pass_a(eib, knp, qp, ke, a)
    m = _combine_max(m_all).reshape(NPAD)
    ev, d_all = _pass_a2(dstp, lg, m)
    (acc,) = _pass_b(dstp, ev, kv)
    recip = _combine_recip(d_all).reshape(NPAD, 1)
    return _norm_relu(acc, recip)


def kernel(x, edge_index, edge_attr,
           Wq0, bq0, Wn0, bn0, We0, be0, a0,
           Wq1, bq1, Wn1, bn1, We1, be1, a1):
    npadE = EPAD - N_EDGES
    pad_ei = jnp.concatenate(
        [jnp.zeros((1, npadE), jnp.int32),
         jnp.full((1, npadE), NPAD - 1, jnp.int32)], axis=0)
    eip = jnp.concatenate([edge_index, pad_ei], axis=1)
    dstp = eip[1]
    eib = jnp.transpose(eip.reshape(2, EPAD // CHA, CHA), (1, 0, 2))
    eap = jnp.concatenate(
        [edge_attr, jnp.zeros((npadE, DE), jnp.float32)], axis=0)
    xp = jnp.concatenate(
        [x, jnp.zeros((NPAD - N_NODES, H), jnp.float32)], axis=0)

    h = _hop(xp, eap, eib, dstp, Wq0, bq0, Wn0, bn0, We0, be0, a0)
    h = _hop(h, eap, eib, dstp, Wq1, bq1, Wn1, bn1, We1, be1, a1)
    return h[:N_NODES]


# speedup vs baseline: 1.0891x; 1.0891x over previous
import functools

import jax
import jax.numpy as jnp
from jax import lax
from jax.experimental import pallas as pl
from jax.experimental.pallas import tpu as pltpu
from jax.experimental.pallas import tpu_sc as plsc

N_NODES = 10000
N_EDGES = 320000
H = 128
DE = 16

NC = 2
NS = 16
NW = NC * NS
LANE = 16

NPAD = 10240
EPAD = 327680
EPW = EPAD // NW
CHA = 64
NCHA = EPW // CHA
CHB = 128
NCHB = EPW // CHB
NEG = -3.0e38

_sc_mesh = plsc.VectorSubcoreMesh(
    core_axis_name="c", subcore_axis_name="s", num_cores=NC, num_subcores=NS)

_sc_params = pltpu.CompilerParams(needs_layout_passes=False)


def _node_mm_body(x_ref, wq_ref, bq_ref, wn_ref, bn_ref, q_ref, kn_ref):
    x = x_ref[...]
    q_ref[...] = jnp.dot(x, wq_ref[...], preferred_element_type=jnp.float32) + bq_ref[...]
    kn_ref[...] = jnp.dot(x, wn_ref[...], preferred_element_type=jnp.float32) + bn_ref[...]


def _node_mm(xp, Wq, bq, Wn, bn):
    blk = 1024
    grid = NPAD // blk
    return pl.pallas_call(
        _node_mm_body,
        grid=(grid,),
        in_specs=[
            pl.BlockSpec((blk, H), lambda i: (i, 0)),
            pl.BlockSpec((H, H), lambda i: (0, 0)),
            pl.BlockSpec((1, H), lambda i: (0, 0)),
            pl.BlockSpec((H, H), lambda i: (0, 0)),
            pl.BlockSpec((1, H), lambda i: (0, 0)),
        ],
        out_specs=[
            pl.BlockSpec((blk, H), lambda i: (i, 0)),
            pl.BlockSpec((blk, H), lambda i: (i, 0)),
        ],
        out_shape=[
            jax.ShapeDtypeStruct((NPAD, H), jnp.float32),
            jax.ShapeDtypeStruct((NPAD, H), jnp.float32),
        ],
    )(xp, Wq, bq.reshape(1, H), Wn, bn.reshape(1, H))


def _edge_mm_body(ea_ref, we_ref, be_ref, ke_ref):
    ke_ref[...] = (jnp.dot(ea_ref[...], we_ref[...],
                           preferred_element_type=jnp.float32) + be_ref[...])


def _edge_mm(eap, We, be):
    blk = 4096
    grid = EPAD // blk
    return pl.pallas_call(
        _edge_mm_body,
        grid=(grid,),
        in_specs=[
            pl.BlockSpec((blk, DE), lambda i: (i, 0)),
            pl.BlockSpec((DE, H), lambda i: (0, 0)),
            pl.BlockSpec((1, H), lambda i: (0, 0)),
        ],
        out_specs=pl.BlockSpec((blk, H), lambda i: (i, 0)),
        out_shape=jax.ShapeDtypeStruct((EPAD, H), jnp.float32),
    )(eap, We, be.reshape(1, H))


def _combine_max_body(mall_ref, m_ref):
    m = jnp.max(mall_ref[...], axis=0, keepdims=True)
    m_ref[...] = jnp.where(m > NEG * 0.5, m, 0.0)


def _combine_max(m_all):
    return pl.pallas_call(
        _combine_max_body,
        out_shape=jax.ShapeDtypeStruct((1, NPAD), jnp.float32),
    )(m_all)


def _combine_recip_body(dall_ref, r_ref):
    d = jnp.sum(dall_ref[...], axis=0, keepdims=True)
    r_ref[...] = 1.0 / (d + 1e-9)


def _combine_recip(d_all):
    return pl.pallas_call(
        _combine_recip_body,
        out_shape=jax.ShapeDtypeStruct((1, NPAD), jnp.float32),
    )(d_all)


def _norm_relu_body(acc_ref, r_ref, out_ref):
    out_ref[...] = jnp.maximum((acc_ref[0] + acc_ref[1]) * r_ref[...], 0.0)


def _norm_relu(acc, recip_col):
    blk = 1024
    grid = NPAD // blk
    return pl.pallas_call(
        _norm_relu_body,
        grid=(grid,),
        in_specs=[
            pl.BlockSpec((NC, blk, H), lambda i: (0, i, 0)),
            pl.BlockSpec((blk, 1), lambda i: (i, 0)),
        ],
        out_specs=pl.BlockSpec((blk, H), lambda i: (i, 0)),
        out_shape=jax.ShapeDtypeStruct((NPAD, H), jnp.float32),
    )(acc, recip_col)


def _wid():
    return lax.axis_index("s") * NC + lax.axis_index("c")


def _scatter_max(tbl, dst16, val16):
    def _pending(_):
        cur = plsc.load_gather(tbl, [dst16])
        return plsc.all_reduce_population_count(val16 > cur)[0] > 0

    def _body(_):
        cur = plsc.load_gather(tbl, [dst16])
        plsc.store_scatter(tbl, [dst16], jnp.maximum(cur, val16),
                           mask=val16 > cur)
        return 0

    lax.while_loop(_pending, _body, 0)


def _sc_pass_a(src_h, dst_h, kn_h, q_h, ke_h, a_h,
               kv_h, lg_h, mall_h,
               a_v, ei_v, kn_v, q_v, ke_v, kv_v, lg_v, m_tbl,
               sem_i, sem_g0, sem_g1, sem_w0, sem_w1):
    w = _wid()
    base0 = w * EPW
    sem_g = (sem_g0, sem_g1)
    sem_w = (sem_w0, sem_w1)
    pltpu.sync_copy(a_h, a_v)

    @pl.loop(0, NPAD // LANE)
    def _init(i):
        m_tbl[pl.ds(i * LANE, LANE)] = jnp.full((LANE,), NEG, jnp.float32)

    def _issue_gathers(ci, b):
        base = base0 + ci * CHA
        pltpu.async_copy(kn_h.at[ei_v.at[b, 0]], kn_v.at[b], sem_g[b])
        pltpu.async_copy(q_h.at[ei_v.at[b, 1]], q_v.at[b], sem_g[b])
        pltpu.async_copy(ke_h.at[pl.ds(base, CHA)], ke_v.at[b], sem_g[b])

    def _wait_gathers(b):
        pltpu.make_async_copy(kn_h.at[pl.ds(0, CHA)], kn_v.at[b], sem_g[b]).wait()
        pltpu.make_async_copy(q_h.at[pl.ds(0, CHA)], q_v.at[b], sem_g[b]).wait()
        pltpu.make_async_copy(ke_h.at[pl.ds(0, CHA)], ke_v.at[b], sem_g[b]).wait()

    def _wait_wb(b):
        pltpu.make_async_copy(kv_v.at[b], kv_h.at[pl.ds(0, CHA)], sem_w[b]).wait()
        pltpu.make_async_copy(lg_v.at[b], lg_h.at[pl.ds(0, CHA)], sem_w[b]).wait()

    def _compute(ci, b):
        base = base0 + ci * CHA

        @pl.loop(0, CHA // LANE)
        def _grp(g):
            lanes = lax.iota(jnp.int32, LANE)
            lg16 = jnp.zeros((LANE,), jnp.float32)
            for k in range(LANE):
                e = g * LANE + k
                acc = jnp.zeros((LANE,), jnp.float32)
                for gg in range(H // LANE):
                    sl = pl.ds(gg * LANE, LANE)
                    kv = kn_v[b, e, sl] + ke_v[b, e, sl]
                    kv_v[b, e, sl] = kv
                    u = q_v[b, e, sl] + kv
                    f = jnp.where(u > 0.0, u, 0.2 * u)
                    acc = acc + f * a_v[sl]
                lg16 = jnp.where(lanes == k, jnp.sum(acc), lg16)
            lg_v[b, pl.ds(g * LANE, LANE)] = lg16
            dst16 = ei_v[b, 1, pl.ds(g * LANE, LANE)]
            _scatter_max(m_tbl, dst16, lg16)

        pltpu.async_copy(kv_v.at[b], kv_h.at[pl.ds(base, CHA)], sem_w[b])
        pltpu.async_copy(lg_v.at[b], lg_h.at[pl.ds(base, CHA)], sem_w[b])

    pltpu.sync_copy(src_h.at[pl.ds(base0, CHA)], ei_v.at[0, 0])
    pltpu.sync_copy(dst_h.at[pl.ds(base0, CHA)], ei_v.at[0, 1])
    _issue_gathers(0, 0)

    @pl.loop(0, NCHA // 2)
    def _outer(g2):
        for b in (0, 1):
            ci = g2 * 2 + b

            @pl.when(ci + 1 < NCHA)
            def _pf():
                nb = 1 - b
                nbase = base0 + (ci + 1) * CHA
                pltpu.sync_copy(src_h.at[pl.ds(nbase, CHA)], ei_v.at[nb, 0])
                pltpu.sync_copy(dst_h.at[pl.ds(nbase, CHA)], ei_v.at[nb, 1])

            @pl.when(ci >= 2)
            def _ww():
                _wait_wb(b)

            _wait_gathers(b)

            @pl.when(ci + 1 < NCHA)
            def _ig():
                _issue_gathers(ci + 1, 1 - b)

            _compute(ci, b)

    _wait_wb(0)
    _wait_wb(1)
    pltpu.sync_copy(m_tbl, mall_h.at[w])


def _sc_pass_b(dst_h, ev_h, kv_h,
               acc_h,
               dst_v, ev_v, kv_v, acc_sh,
               sem_s0, sem_s1, sem_a0, sem_a1):
    c = lax.axis_index("c")
    s = lax.axis_index("s")
    w = _wid()
    base0 = w * EPW
    rows_per_tile = NPAD // NS
    sem_s = (sem_s0, sem_s1)
    sem_a = (sem_a0, sem_a1)

    @pl.loop(0, CHB)
    def _z(e):
        for g in range(H // LANE):
            kv_v[0, e, pl.ds(g * LANE, LANE)] = jnp.zeros((LANE,), jnp.float32)

    for j in range(rows_per_tile // CHB):
        pltpu.sync_copy(kv_v.at[0],
                        acc_sh.at[pl.ds(s * rows_per_tile + j * CHB, CHB)])

    plsc.subcore_barrier()

    def _issue_streams(ci, b):
        base = base0 + ci * CHB
        pltpu.async_copy(dst_h.at[pl.ds(base, CHB)], dst_v.at[b], sem_s[b])
        pltpu.async_copy(ev_h.at[pl.ds(base, CHB)], ev_v.at[b], sem_s[b])
        pltpu.async_copy(kv_h.at[pl.ds(base, CHB)], kv_v.at[b], sem_s[b])

    def _wait_streams(b):
        pltpu.make_async_copy(dst_h.at[pl.ds(0, CHB)], dst_v.at[b], sem_s[b]).wait()
        pltpu.make_async_copy(ev_h.at[pl.ds(0, CHB)], ev_v.at[b], sem_s[b]).wait()
        pltpu.make_async_copy(kv_h.at[pl.ds(0, CHB)], kv_v.at[b], sem_s[b]).wait()

    def _wait_scatter(b):
        pltpu.make_async_copy(kv_v.at[b], acc_sh.at[pl.ds(0, CHB)],
                              sem_a[b]).wait()

    _issue_streams(0, 0)

    @pl.loop(0, NCHB // 2)
    def _outer(g2):
        for b in (0, 1):
            ci = g2 * 2 + b

            @pl.when(ci >= 1)
            def _wa():
                _wait_scatter(1 - b)

            @pl.when(ci + 1 < NCHB)
            def _pf():
                _issue_streams(ci + 1, 1 - b)

            _wait_streams(b)

            @pl.loop(0, CHB // LANE)
            def _scale(g):
                ev16 = ev_v[b, pl.ds(g * LANE, LANE)]
                for k in range(LANE):
                    e = g * LANE + k
                    a_s = ev16[k]
                    for gg in range(H // LANE):
                        ssl = pl.ds(gg * LANE, LANE)
                        kv_v[b, e, ssl] = kv_v[b, e, ssl] * a_s

            pltpu.async_copy(kv_v.at[b], acc_sh.at[dst_v.at[b]], sem_a[b],
                             add=True)

    _wait_scatter((NCHB - 1) % 2)
    plsc.subcore_barrier()

    for j in range(rows_per_tile // CHB):
        r0 = s * rows_per_tile + j * CHB
        pltpu.sync_copy(acc_sh.at[pl.ds(r0, CHB)], kv_v.at[0])
        pltpu.sync_copy(kv_v.at[0], acc_h.at[c].at[pl.ds(r0, CHB)])


def _sc_pass_a2(dst_h, lg_h, m_h,
                ev_h, dall_h,
                m_tbl, den_tbl, dst_v, lg_v, ev_v):
    w = _wid()
    base0 = w * EPW
    CH2 = 1024
    pltpu.sync_copy(m_h, m_tbl)

    @pl.loop(0, NPAD // LANE)
    def _init(i):
        den_tbl[pl.ds(i * LANE, LANE)] = jnp.zeros((LANE,), jnp.float32)

    @pl.loop(0, EPW // CH2)
    def _chunk(ci):
        base = base0 + ci * CH2
        pltpu.sync_copy(dst_h.at[pl.ds(base, CH2)], dst_v)
        pltpu.sync_copy(lg_h.at[pl.ds(base, CH2)], lg_v)

        @pl.loop(0, CH2 // LANE)
        def _vec(g):
            sl = pl.ds(g * LANE, LANE)
            dv = dst_v[sl]
            mv = plsc.load_gather(m_tbl, [dv])
            ev = jnp.exp(lg_v[sl] - mv)
            ev_v[sl] = ev
            plsc.addupdate_scatter(den_tbl, [dv], ev)

        pltpu.sync_copy(ev_v, ev_h.at[pl.ds(base, CH2)])

    pltpu.sync_copy(den_tbl, dall_h.at[w])


_pass_a = pl.kernel(
    _sc_pass_a,
    out_type=[
        jax.ShapeDtypeStruct((EPAD, H), jnp.float32),
        jax.ShapeDtypeStruct((EPAD,), jnp.float32),
        jax.ShapeDtypeStruct((NW, NPAD), jnp.float32),
    ],
    mesh=_sc_mesh,
    compiler_params=_sc_params,
    scratch_types=[
        pltpu.VMEM((H,), jnp.float32),
        pltpu.VMEM((2, 2, CHA), jnp.int32),
        pltpu.VMEM((2, CHA, H), jnp.float32),
        pltpu.VMEM((2, CHA, H), jnp.float32),
        pltpu.VMEM((2, CHA, H), jnp.float32),
        pltpu.VMEM((2, CHA, H), jnp.float32),
        pltpu.VMEM((2, CHA), jnp.float32),
        pltpu.VMEM((NPAD,), jnp.float32),
        pltpu.SemaphoreType.DMA,
        pltpu.SemaphoreType.DMA,
        pltpu.SemaphoreType.DMA,
        pltpu.SemaphoreType.DMA,
        pltpu.SemaphoreType.DMA,
    ],
)

_pass_b = pl.kernel(
    _sc_pass_b,
    out_type=[
        jax.ShapeDtypeStruct((NC, NPAD, H), jnp.float32),
    ],
    mesh=_sc_mesh,
    compiler_params=_sc_params,
    scratch_types=[
        pltpu.VMEM((2, CHB), jnp.int32),
        pltpu.VMEM((2, CHB), jnp.float32),
        pltpu.VMEM((2, CHB, H), jnp.float32),
        pltpu.VMEM_SHARED((NPAD, H), jnp.float32),
        pltpu.SemaphoreType.DMA,
        pltpu.SemaphoreType.DMA,
        pltpu.SemaphoreType.DMA,
        pltpu.SemaphoreType.DMA,
    ],
)

_pass_a2 = pl.kernel(
    _sc_pass_a2,
    out_type=[
        jax.ShapeDtypeStruct((EPAD,), jnp.float32),
        jax.ShapeDtypeStruct((NW, NPAD), jnp.float32),
    ],
    mesh=_sc_mesh,
    compiler_params=_sc_params,
    scratch_types=[
        pltpu.VMEM((NPAD,), jnp.float32),
        pltpu.VMEM((NPAD,), jnp.float32),
        pltpu.VMEM((1024,), jnp.int32),
        pltpu.VMEM((1024,), jnp.float32),
        pltpu.VMEM((1024,), jnp.float32),
    ],
)


def _hop(xp, eap, srcp, dstp, Wq, bq, Wn, bn, We, be, a):
    qp, knp = _node_mm(xp, Wq, bq, Wn, bn)
    ke = _edge_mm(eap, We, be)
    kv, lg, m_all = _pass_a(srcp, dstp, knp, qp, ke, a)
    m = _combine_max(m_all).reshape(NPAD)
    ev, d_all = _pass_a2(dstp, lg, m)
    (acc,) = _pass_b(dstp, ev, kv)
    recip = _combine_recip(d_all).reshape(NPAD, 1)
    return _norm_relu(acc, recip)


def kernel(x, edge_index, edge_attr,
           Wq0, bq0, Wn0, bn0, We0, be0, a0,
           Wq1, bq1, Wn1, bn1, We1, be1, a1):
    npadE = EPAD - N_EDGES
    pad_ei = jnp.concatenate(
        [jnp.zeros((1, npadE), jnp.int32),
         jnp.full((1, npadE), NPAD - 1, jnp.int32)], axis=0)
    eip = jnp.concatenate([edge_index, pad_ei], axis=1)
    srcp = eip[0]
    dstp = eip[1]
    eap = jnp.concatenate(
        [edge_attr, jnp.zeros((npadE, DE), jnp.float32)], axis=0)
    xp = jnp.concatenate(
        [x, jnp.zeros((NPAD - N_NODES, H), jnp.float32)], axis=0)

    h = _hop(xp, eap, srcp, dstp, Wq0, bq0, Wn0, bn0, We0, be0, a0)
    h = _hop(h, eap, srcp, dstp, Wq1, bq1, Wn1, bn1, We1, be1, a1)
    return h[:N_NODES]


# speedup vs baseline: 1.3244x; 1.2160x over previous
import functools

import jax
import jax.numpy as jnp
from jax import lax
from jax.experimental import pallas as pl
from jax.experimental.pallas import tpu as pltpu
from jax.experimental.pallas import tpu_sc as plsc

N_NODES = 10000
N_EDGES = 320000
H = 128
DE = 16

NC = 2
NS = 16
NW = NC * NS
LANE = 16

NPAD = 10240
EPAD = 327680
EPW = EPAD // NW
CHA = 64
NCHA = EPW // CHA
CHB = 128
NCHB = EPW // CHB
NEG = -3.0e38

_sc_mesh = plsc.VectorSubcoreMesh(
    core_axis_name="c", subcore_axis_name="s", num_cores=NC, num_subcores=NS)

_sc_params = pltpu.CompilerParams(needs_layout_passes=False)


def _node_mm_body(x_ref, wq_ref, bq_ref, wn_ref, bn_ref, q_ref, kn_ref):
    x = x_ref[...]
    q_ref[...] = jnp.dot(x, wq_ref[...], preferred_element_type=jnp.float32) + bq_ref[...]
    kn_ref[...] = jnp.dot(x, wn_ref[...], preferred_element_type=jnp.float32) + bn_ref[...]


def _node_mm(xp, Wq, bq, Wn, bn):
    blk = 1024
    grid = NPAD // blk
    return pl.pallas_call(
        _node_mm_body,
        grid=(grid,),
        in_specs=[
            pl.BlockSpec((blk, H), lambda i: (i, 0)),
            pl.BlockSpec((H, H), lambda i: (0, 0)),
            pl.BlockSpec((1, H), lambda i: (0, 0)),
            pl.BlockSpec((H, H), lambda i: (0, 0)),
            pl.BlockSpec((1, H), lambda i: (0, 0)),
        ],
        out_specs=[
            pl.BlockSpec((blk, H), lambda i: (i, 0)),
            pl.BlockSpec((blk, H), lambda i: (i, 0)),
        ],
        out_shape=[
            jax.ShapeDtypeStruct((NPAD, H), jnp.float32),
            jax.ShapeDtypeStruct((NPAD, H), jnp.float32),
        ],
    )(xp, Wq, bq.reshape(1, H), Wn, bn.reshape(1, H))


def _edge_mm_body(ea_ref, we_ref, be_ref, ke_ref):
    ke_ref[...] = (jnp.dot(ea_ref[...], we_ref[...],
                           preferred_element_type=jnp.float32) + be_ref[...])


def _edge_mm(eap, We, be):
    blk = 4096
    grid = EPAD // blk
    return pl.pallas_call(
        _edge_mm_body,
        grid=(grid,),
        in_specs=[
            pl.BlockSpec((blk, DE), lambda i: (i, 0)),
            pl.BlockSpec((DE, H), lambda i: (0, 0)),
            pl.BlockSpec((1, H), lambda i: (0, 0)),
        ],
        out_specs=pl.BlockSpec((blk, H), lambda i: (i, 0)),
        out_shape=jax.ShapeDtypeStruct((EPAD, H), jnp.float32),
    )(eap, We, be.reshape(1, H))


def _combine_max_body(mall_ref, m_ref):
    m = jnp.max(mall_ref[...], axis=0, keepdims=True)
    m_ref[...] = jnp.where(m > NEG * 0.5, m, 0.0)


def _combine_max(m_all):
    return pl.pallas_call(
        _combine_max_body,
        out_shape=jax.ShapeDtypeStruct((1, NPAD), jnp.float32),
    )(m_all)


def _combine_recip_body(dall_ref, r_ref):
    d = jnp.sum(dall_ref[...], axis=0, keepdims=True)
    r_ref[...] = 1.0 / (d + 1e-9)


def _combine_recip(d_all):
    return pl.pallas_call(
        _combine_recip_body,
        out_shape=jax.ShapeDtypeStruct((1, NPAD), jnp.float32),
    )(d_all)


def _norm_relu_body(acc_ref, r_ref, out_ref):
    out_ref[...] = jnp.maximum((acc_ref[0] + acc_ref[1]) * r_ref[...], 0.0)


def _norm_relu(acc, recip_col):
    blk = 1024
    grid = NPAD // blk
    return pl.pallas_call(
        _norm_relu_body,
        grid=(grid,),
        in_specs=[
            pl.BlockSpec((NC, blk, H), lambda i: (0, i, 0)),
            pl.BlockSpec((blk, 1), lambda i: (i, 0)),
        ],
        out_specs=pl.BlockSpec((blk, H), lambda i: (i, 0)),
        out_shape=jax.ShapeDtypeStruct((NPAD, H), jnp.float32),
    )(acc, recip_col)


def _wid():
    return lax.axis_index("s") * NC + lax.axis_index("c")


def _scatter_max(tbl, dst16, val16):
    def _pending(_):
        cur = plsc.load_gather(tbl, [dst16])
        return plsc.all_reduce_population_count(val16 > cur)[0] > 0

    def _body(_):
        cur = plsc.load_gather(tbl, [dst16])
        plsc.store_scatter(tbl, [dst16], jnp.maximum(cur, val16),
                           mask=val16 > cur)
        return 0

    lax.while_loop(_pending, _body, 0)


def _sc_pass_a(src_h, dst_h, kn_h, q_h, ke_h, a_h,
               kv_h, lg_h, mall_h,
               a_v, ei_v, kn_v, q_v, ke_v, kv_v, lg_v, m_tbl,
               sem_i, sem_g0, sem_g1, sem_w0, sem_w1):
    w = _wid()
    base0 = w * EPW
    sem_g = (sem_g0, sem_g1)
    sem_w = (sem_w0, sem_w1)
    pltpu.sync_copy(a_h, a_v)

    @pl.loop(0, NPAD // LANE)
    def _init(i):
        m_tbl[pl.ds(i * LANE, LANE)] = jnp.full((LANE,), NEG, jnp.float32)

    def _issue_gathers(ci, b):
        base = base0 + ci * CHA
        pltpu.async_copy(kn_h.at[ei_v.at[b, 0]], kn_v.at[b], sem_g[b])
        pltpu.async_copy(q_h.at[ei_v.at[b, 1]], q_v.at[b], sem_g[b])
        pltpu.async_copy(ke_h.at[pl.ds(base, CHA)], ke_v.at[b], sem_g[b])

    def _wait_gathers(b):
        pltpu.make_async_copy(kn_h.at[pl.ds(0, CHA)], kn_v.at[b], sem_g[b]).wait()
        pltpu.make_async_copy(q_h.at[pl.ds(0, CHA)], q_v.at[b], sem_g[b]).wait()
        pltpu.make_async_copy(ke_h.at[pl.ds(0, CHA)], ke_v.at[b], sem_g[b]).wait()

    def _wait_wb(b):
        pltpu.make_async_copy(kv_v.at[b], kv_h.at[pl.ds(0, CHA)], sem_w[b]).wait()
        pltpu.make_async_copy(lg_v.at[b], lg_h.at[pl.ds(0, CHA)], sem_w[b]).wait()

    def _compute(ci, b):
        base = base0 + ci * CHA

        @pl.loop(0, CHA // LANE)
        def _grp(g):
            lanes = lax.iota(jnp.int32, LANE)
            lg16 = jnp.zeros((LANE,), jnp.float32)
            for k in range(LANE):
                e = g * LANE + k
                acc = jnp.zeros((LANE,), jnp.float32)
                for gg in range(H // LANE):
                    sl = pl.ds(gg * LANE, LANE)
                    kv = kn_v[b, e, sl] + ke_v[b, e, sl]
                    kv_v[b, e, sl] = kv
                    u = q_v[b, e, sl] + kv
                    f = jnp.where(u > 0.0, u, 0.2 * u)
                    acc = acc + f * a_v[sl]
                lg16 = jnp.where(lanes == k, jnp.sum(acc), lg16)
            lg_v[b, pl.ds(g * LANE, LANE)] = lg16
            dst16 = ei_v[b, 1, pl.ds(g * LANE, LANE)]
            _scatter_max(m_tbl, dst16, lg16)

        pltpu.async_copy(kv_v.at[b], kv_h.at[pl.ds(base, CHA)], sem_w[b])
        pltpu.async_copy(lg_v.at[b], lg_h.at[pl.ds(base, CHA)], sem_w[b])

    pltpu.sync_copy(src_h.at[pl.ds(base0, CHA)], ei_v.at[0, 0])
    pltpu.sync_copy(dst_h.at[pl.ds(base0, CHA)], ei_v.at[0, 1])
    _issue_gathers(0, 0)

    @pl.loop(0, NCHA // 2)
    def _outer(g2):
        for b in (0, 1):
            ci = g2 * 2 + b

            @pl.when(ci + 1 < NCHA)
            def _pf():
                nb = 1 - b
                nbase = base0 + (ci + 1) * CHA
                pltpu.sync_copy(src_h.at[pl.ds(nbase, CHA)], ei_v.at[nb, 0])
                pltpu.sync_copy(dst_h.at[pl.ds(nbase, CHA)], ei_v.at[nb, 1])

            @pl.when(ci >= 2)
            def _ww():
                _wait_wb(b)

            _wait_gathers(b)

            @pl.when(ci + 1 < NCHA)
            def _ig():
                _issue_gathers(ci + 1, 1 - b)

            _compute(ci, b)

    _wait_wb(0)
    _wait_wb(1)
    pltpu.sync_copy(m_tbl, mall_h.at[w])


def _sc_pass_b(dst_h, ev_h, kv_h,
               acc_h,
               dst_v, ev_v, kv_v, acc_sh,
               sem_s0, sem_s1, sem_a0, sem_a1):
    c = lax.axis_index("c")
    s = lax.axis_index("s")
    w = _wid()
    base0 = w * EPW
    rows_per_tile = NPAD // NS
    sem_s = (sem_s0, sem_s1)
    sem_a = (sem_a0, sem_a1)

    @pl.loop(0, CHB)
    def _z(e):
        for g in range(H // LANE):
            kv_v[0, e, pl.ds(g * LANE, LANE)] = jnp.zeros((LANE,), jnp.float32)

    for j in range(rows_per_tile // CHB):
        pltpu.sync_copy(kv_v.at[0],
                        acc_sh.at[pl.ds(s * rows_per_tile + j * CHB, CHB)])

    plsc.subcore_barrier()

    def _issue_streams(ci, b):
        base = base0 + ci * CHB
        pltpu.async_copy(dst_h.at[pl.ds(base, CHB)], dst_v.at[b], sem_s[b])
        pltpu.async_copy(ev_h.at[pl.ds(base, CHB)], ev_v.at[b], sem_s[b])
        pltpu.async_copy(kv_h.at[pl.ds(base, CHB)], kv_v.at[b], sem_s[b])

    def _wait_streams(b):
        pltpu.make_async_copy(dst_h.at[pl.ds(0, CHB)], dst_v.at[b], sem_s[b]).wait()
        pltpu.make_async_copy(ev_h.at[pl.ds(0, CHB)], ev_v.at[b], sem_s[b]).wait()
        pltpu.make_async_copy(kv_h.at[pl.ds(0, CHB)], kv_v.at[b], sem_s[b]).wait()

    def _wait_scatter(b):
        pltpu.make_async_copy(kv_v.at[b], acc_sh.at[pl.ds(0, CHB)],
                              sem_a[b]).wait()

    _issue_streams(0, 0)

    @pl.loop(0, NCHB // 2)
    def _outer(g2):
        for b in (0, 1):
            ci = g2 * 2 + b

            @pl.when(ci >= 1)
            def _wa():
                _wait_scatter(1 - b)

            @pl.when(ci + 1 < NCHB)
            def _pf():
                _issue_streams(ci + 1, 1 - b)

            _wait_streams(b)

            @pl.loop(0, CHB // LANE)
            def _scale(g):
                ev16 = ev_v[b, pl.ds(g * LANE, LANE)]
                for k in range(LANE):
                    e = g * LANE + k
                    a_s = ev16[k]
                    for gg in range(H // LANE):
                        ssl = pl.ds(gg * LANE, LANE)
                        kv_v[b, e, ssl] = kv_v[b, e, ssl] * a_s

            pltpu.async_copy(kv_v.at[b], acc_sh.at[dst_v.at[b]], sem_a[b],
                             add=True)

    _wait_scatter((NCHB - 1) % 2)
    plsc.subcore_barrier()

    for j in range(rows_per_tile // CHB):
        r0 = s * rows_per_tile + j * CHB
        pltpu.sync_copy(acc_sh.at[pl.ds(r0, CHB)], kv_v.at[0])
        pltpu.sync_copy(kv_v.at[0], acc_h.at[c].at[pl.ds(r0, CHB)])


def _sc_pass_a2(dst_h, lg_h, m_h,
                ev_h, dall_h,
                m_tbl, den_tbl, dst_v, lg_v, ev_v):
    w = _wid()
    base0 = w * EPW
    CH2 = 1024
    pltpu.sync_copy(m_h, m_tbl)

    @pl.loop(0, NPAD // LANE)
    def _init(i):
        den_tbl[pl.ds(i * LANE, LANE)] = jnp.zeros((LANE,), jnp.float32)

    @pl.loop(0, EPW // CH2)
    def _chunk(ci):
        base = base0 + ci * CH2
        pltpu.sync_copy(dst_h.at[pl.ds(base, CH2)], dst_v)
        pltpu.sync_copy(lg_h.at[pl.ds(base, CH2)], lg_v)

        @pl.loop(0, CH2 // LANE)
        def _vec(g):
            sl = pl.ds(g * LANE, LANE)
            dv = dst_v[sl]
            mv = plsc.load_gather(m_tbl, [dv])
            ev = jnp.exp(lg_v[sl] - mv)
            ev_v[sl] = ev
            plsc.addupdate_scatter(den_tbl, [dv], ev)

        pltpu.sync_copy(ev_v, ev_h.at[pl.ds(base, CH2)])

    pltpu.sync_copy(den_tbl, dall_h.at[w])


_pass_a = pl.kernel(
    _sc_pass_a,
    out_type=[
        jax.ShapeDtypeStruct((EPAD, H), jnp.float32),
        jax.ShapeDtypeStruct((EPAD,), jnp.float32),
        jax.ShapeDtypeStruct((NW, NPAD), jnp.float32),
    ],
    mesh=_sc_mesh,
    compiler_params=_sc_params,
    scratch_types=[
        pltpu.VMEM((H,), jnp.float32),
        pltpu.VMEM((2, 2, CHA), jnp.int32),
        pltpu.VMEM((2, CHA, H), jnp.float32),
        pltpu.VMEM((2, CHA, H), jnp.float32),
        pltpu.VMEM((2, CHA, H), jnp.float32),
        pltpu.VMEM((2, CHA, H), jnp.float32),
        pltpu.VMEM((2, CHA), jnp.float32),
        pltpu.VMEM((NPAD,), jnp.float32),
        pltpu.SemaphoreType.DMA,
        pltpu.SemaphoreType.DMA,
        pltpu.SemaphoreType.DMA,
        pltpu.SemaphoreType.DMA,
        pltpu.SemaphoreType.DMA,
    ],
)

_pass_b = pl.kernel(
    _sc_pass_b,
    out_type=[
        jax.ShapeDtypeStruct((NC, NPAD, H), jnp.float32),
    ],
    mesh=_sc_mesh,
    compiler_params=_sc_params,
    scratch_types=[
        pltpu.VMEM((2, CHB), jnp.int32),
        pltpu.VMEM((2, CHB), jnp.float32),
        pltpu.VMEM((2, CHB, H), jnp.float32),
        pltpu.VMEM_SHARED((NPAD, H), jnp.float32),
        pltpu.SemaphoreType.DMA,
        pltpu.SemaphoreType.DMA,
        pltpu.SemaphoreType.DMA,
        pltpu.SemaphoreType.DMA,
    ],
)

_pass_a2 = pl.kernel(
    _sc_pass_a2,
    out_type=[
        jax.ShapeDtypeStruct((EPAD,), jnp.float32),
        jax.ShapeDtypeStruct((NW, NPAD), jnp.float32),
    ],
    mesh=_sc_mesh,
    compiler_params=_sc_params,
    scratch_types=[
        pltpu.VMEM((NPAD,), jnp.float32),
        pltpu.VMEM((NPAD,), jnp.float32),
        pltpu.VMEM((1024,), jnp.int32),
        pltpu.VMEM((1024,), jnp.float32),
        pltpu.VMEM((1024,), jnp.float32),
    ],
)


def _hop(xp, eap, srcp, dstp, Wq, bq, Wn, bn, We, be, a):
    qp, knp = _node_mm(xp, Wq, bq, Wn, bn)
    ke = _edge_mm(eap, We, be)
    kv, lg, m_all = _pass_a(srcp, dstp, knp, qp, ke, a)
    m = _combine_max(m_all).reshape(NPAD)
    ev, d_all = _pass_a2(dstp, lg, m)
    (acc,) = _pass_b(dstp, ev, kv)
    recip = _combine_recip(d_all).reshape(NPAD, 1)
    return _norm_relu(acc, recip)


def kernel(x, edge_index, edge_attr,
           Wq0, bq0, Wn0, bn0, We0, be0, a0,
           Wq1, bq1, Wn1, bn1, We1, be1, a1):
    npadE = EPAD - N_EDGES
    pad_iota = jnp.arange(npadE, dtype=jnp.int32)
    pad_ei = jnp.stack(
        [pad_iota % N_NODES,
         N_NODES + pad_iota % (NPAD - N_NODES)], axis=0)
    eip = jnp.concatenate([edge_index, pad_ei], axis=1)
    srcp = eip[0]
    dstp = eip[1]
    eap = jnp.concatenate(
        [edge_attr, jnp.zeros((npadE, DE), jnp.float32)], axis=0)
    xp = jnp.concatenate(
        [x, jnp.zeros((NPAD - N_NODES, H), jnp.float32)], axis=0)

    h = _hop(xp, eap, srcp, dstp, Wq0, bq0, Wn0, bn0, We0, be0, a0)
    h = _hop(h, eap, srcp, dstp, Wq1, bq1, Wn1, bn1, We1, be1, a1)
    return h[:N_NODES]
